# Initial kernel scaffold; baseline (speedup 1.0000x reference)
import jax
import jax.numpy as jnp
from jax.experimental import pallas as pl


def kernel(b1, b2, b3, b4, corr, zf_ori, template_mask, w_b4, bias_b4, w_b3, bias_b3, w_b2, bias_b2, w_b1, bias_b1, w_rCo, bias_rCo, w_r3, bias_r3, w_r2, bias_r2, w_r1, bias_r1, w_r0, bias_r0, w_s_embed, bias_s_embed, w_t_embed, bias_t_embed):
    raise NotImplementedError("write your pallas kernel here")


# speedup vs baseline: 3.1919x; 3.1919x over previous
import math

import numpy as np
import jax
import jax.numpy as jnp
from jax.experimental import pallas as pl
from jax.experimental.pallas import tpu as pltpu

_NB = 32


def _interp_1d(n_out, n_in):
    if n_out == 1:
        src = np.zeros((1,), np.float64)
    else:
        src = np.arange(n_out, dtype=np.float64) * (n_in - 1) / (n_out - 1)
    lo = np.clip(np.floor(src).astype(np.int64), 0, n_in - 1)
    hi = np.minimum(lo + 1, n_in - 1)
    frac = (src - lo).astype(np.float32)
    op = np.zeros((n_out, n_in), np.float32)
    op[np.arange(n_out), lo] += 1.0 - frac
    op[np.arange(n_out), hi] += frac
    return op


def _resize_op(out_hw, in_hw):
    return np.kron(_interp_1d(out_hw[0], in_hw[0]),
                   _interp_1d(out_hw[1], in_hw[1])).astype(np.float32)


def _shift_ops(hw):
    H, W = hw
    n = H * W
    ops = np.zeros((9, n, n), np.float32)
    for kh in (-1, 0, 1):
        for kw in (-1, 0, 1):
            blk = np.zeros((H, W, H, W), np.float32)
            for h in range(max(0, -kh), min(H, H - kh)):
                for w in range(max(0, -kw), min(W, W - kw)):
                    blk[h, w, h + kh, w + kw] = 1.0
            ops[(kh + 1) * 3 + (kw + 1)] = blk.reshape(n, n)
    return ops


def _mss_kernel(*refs):
    (b4_ref, zf_ref, msk_ref, corr_ref, b3_ref, b2_ref, b1_ref,
     rmt_ref, ws_ref, bs_ref, wt_ref, tb_ref) = refs[:12]
    out_ref = refs[-1]
    cref = refs[12:-1]
    convs = [tuple(cref[3 * i:3 * i + 3]) for i in range(9)]
    c_rco, c_b4, c_r3, c_b3, c_r2, c_b2, c_r1, c_b1, c_r0 = convs

    f32 = jnp.float32
    bf16 = jnp.bfloat16
    nb = b4_ref.shape[0]

    def conv(x, at_r, wc_r, bias_r, relu):
        at = at_r[...]
        nin, nout = at.shape[1], at.shape[2]
        cout = wc_r.shape[1] // 9
        t = jnp.dot(x, wc_r[...], preferred_element_type=f32).astype(bf16)
        t3 = t.reshape(nb, nin, 9 * cout)
        acc = jnp.zeros((nb * cout, nout), f32)
        for k in range(9):
            tk = jnp.transpose(t3[:, :, k * cout:(k + 1) * cout],
                               (0, 2, 1)).reshape(nb * cout, nin)
            acc = acc + jnp.dot(tk, at[k], preferred_element_type=f32)
        acc = acc + bias_r[...]
        if relu:
            acc = jnp.maximum(acc, 0.0)
        return acc

    def to_x(acc_t, cout):
        a3 = acc_t.astype(bf16).reshape(nb, cout, acc_t.shape[1])
        return jnp.transpose(a3, (0, 2, 1)).reshape(-1, cout)

    b4 = b4_ref[...]
    nx, nz = b4.shape[1], zf_ref.shape[1]
    ce = ws_ref.shape[1]
    xe = (jnp.dot(b4.reshape(nb * nx, -1), ws_ref[...],
                  preferred_element_type=f32) + bs_ref[...])
    ze = (jnp.dot(zf_ref[...].reshape(nb * nz, -1), wt_ref[...],
                  preferred_element_type=f32) + tb_ref[...])
    att = jax.lax.dot_general(
        xe.reshape(nb, nx, ce), ze.reshape(nb, nz, ce),
        (((2,), (2,)), ((0,), (0,))),
        preferred_element_type=f32) * (1.0 / math.sqrt(ce))
    att = att - jnp.max(att, axis=-1, keepdims=True)
    p = jnp.exp(att)
    att = p * pl.reciprocal(jnp.sum(p, axis=-1, keepdims=True), approx=True)
    mup = jnp.dot(msk_ref[...], rmt_ref[...],
                  preferred_element_type=f32)
    arn = jnp.sum(att * mup[:, None, :], axis=-1)
    arn = jnp.clip(arn, 0.0, 1.0)
    b4a = (b4.astype(f32) + arn[:, :, None]).astype(bf16).reshape(nb * nx, -1)

    r4 = (conv(corr_ref[...].reshape(nb * corr_ref.shape[1], -1),
               *c_rco, relu=True)
          + conv(b4a, *c_b4, relu=True))
    r3 = (conv(to_x(r4, c_r3[1].shape[0]), *c_r3, relu=True)
          + conv(b3_ref[...].reshape(nb * b3_ref.shape[1], -1),
                 *c_b3, relu=True))
    r2 = (conv(to_x(r3, c_r2[1].shape[0]), *c_r2, relu=True)
          + conv(b2_ref[...].reshape(nb * b2_ref.shape[1], -1),
                 *c_b2, relu=True))
    r1 = (conv(to_x(r2, c_r1[1].shape[0]), *c_r1, relu=True)
          + conv(b1_ref[...].reshape(nb * b1_ref.shape[1], -1),
                 *c_b1, relu=True))
    mask = conv(to_x(r1, c_r0[1].shape[0]), *c_r0, relu=False)
    out_ref[0] = mask.astype(out_ref.dtype)


def kernel(b1, b2, b3, b4, corr, zf_ori, template_mask,
           w_b4, bias_b4, w_b3, bias_b3, w_b2, bias_b2, w_b1, bias_b1,
           w_rCo, bias_rCo, w_r3, bias_r3, w_r2, bias_r2, w_r1, bias_r1,
           w_r0, bias_r0, w_s_embed, bias_s_embed, w_t_embed, bias_t_embed):
    B = b4.shape[0]
    nb = _NB if B % _NB == 0 else (16 if B % 16 == 0 else 1)
    grid = B // nb
    out_hw = (16, 16)
    bf16 = jnp.bfloat16
    f32 = jnp.float32

    hw = lambda t: (int(t.shape[2]), int(t.shape[3]))
    b1_hw, b2_hw, b3_hw, b4_hw = hw(b1), hw(b2), hw(b3), hw(b4)
    corr_hw, zf_hw = hw(corr), hw(zf_ori)
    msk_hw = (int(template_mask.shape[1]), int(template_mask.shape[2]))

    def flat(x):
        Bx, C, H, W = x.shape
        return jnp.transpose(x, (0, 2, 3, 1)).reshape(Bx, H * W, C).astype(bf16)

    b4s, zfs, corrs = flat(b4), flat(zf_ori), flat(corr)
    b3s, b2s, b1s = flat(b3), flat(b2), flat(b1)
    msks = template_mask.reshape(B, msk_hw[0] * msk_hw[1]).astype(f32)

    shift = {s: _shift_ops(s) for s in {b4_hw, b3_hw, b2_hw, b1_hw, out_hw}}
    spatial = {
        'rCo': shift[b4_hw] @ _resize_op(b4_hw, corr_hw),
        'b4':  shift[b4_hw],
        'r3':  shift[b3_hw] @ _resize_op(b3_hw, b4_hw),
        'b3':  shift[b3_hw],
        'r2':  shift[b2_hw] @ _resize_op(b2_hw, b3_hw),
        'b2':  shift[b2_hw],
        'r1':  shift[b1_hw] @ _resize_op(b1_hw, b2_hw),
        'b1':  shift[b1_hw],
        'r0':  shift[out_hw] @ _resize_op(out_hw, b1_hw),
    }
    rm_t = jnp.asarray(_resize_op(zf_hw, msk_hw).T, f32)

    params = {'rCo': (w_rCo, bias_rCo), 'b4': (w_b4, bias_b4),
              'r3': (w_r3, bias_r3), 'b3': (w_b3, bias_b3),
              'r2': (w_r2, bias_r2), 'b2': (w_b2, bias_b2),
              'r1': (w_r1, bias_r1), 'b1': (w_b1, bias_b1),
              'r0': (w_r0, bias_r0)}
    conv_args = []
    for name in ('rCo', 'b4', 'r3', 'b3', 'r2', 'b2', 'r1', 'b1', 'r0'):
        w, bias = params[name]
        cin, cout = int(w.shape[2]), int(w.shape[3])
        wcat = (w.reshape(9, cin, cout).transpose(1, 0, 2)
                .reshape(cin, 9 * cout).astype(bf16))
        a_t = np.ascontiguousarray(spatial[name].transpose(0, 2, 1))
        conv_args += [jnp.asarray(a_t, bf16),
                      wcat,
                      jnp.tile(bias.astype(f32).reshape(cout, 1),
                               (nb, 1))]

    args = [b4s, zfs, msks, corrs, b3s, b2s, b1s,
            rm_t, w_s_embed.astype(bf16), bias_s_embed.astype(f32),
            w_t_embed.astype(bf16), bias_t_embed.astype(f32)] + conv_args

    def batched(shape):
        nz = (0,) * (len(shape) - 1)
        return pl.BlockSpec((nb,) + tuple(shape[1:]),
                            lambda i, _z=nz: (i,) + _z)

    def rep(shape):
        zeros = (0,) * len(shape)
        return pl.BlockSpec(tuple(shape), lambda i, _z=zeros: _z)

    in_specs = ([batched(a.shape) for a in args[:7]]
                + [rep(a.shape) for a in args[7:]])

    n_out = out_hw[0] * out_hw[1]
    cout_final = int(w_r0.shape[3])
    out_shape = jax.ShapeDtypeStruct((grid, nb * cout_final, n_out), f32)

    mask_t = pl.pallas_call(
        _mss_kernel,
        out_shape=out_shape,
        grid=(grid,),
        in_specs=in_specs,
        out_specs=pl.BlockSpec((1, nb * cout_final, n_out),
                               lambda i: (i, 0, 0)),
        compiler_params=pltpu.CompilerParams(
            dimension_semantics=("parallel",),
            vmem_limit_bytes=56 * 1024 * 1024),
    )(*args)

    mask = mask_t.reshape(B, cout_final, out_hw[0], out_hw[1])
    return mask


# speedup vs baseline: 5.2805x; 1.6544x over previous
import math

import numpy as np
import jax
import jax.numpy as jnp
from jax.experimental import pallas as pl
from jax.experimental.pallas import tpu as pltpu

_NB = 32


def _interp_1d(n_out, n_in):
    if n_out == 1:
        src = np.zeros((1,), np.float64)
    else:
        src = np.arange(n_out, dtype=np.float64) * (n_in - 1) / (n_out - 1)
    lo = np.clip(np.floor(src).astype(np.int64), 0, n_in - 1)
    hi = np.minimum(lo + 1, n_in - 1)
    frac = (src - lo).astype(np.float32)
    op = np.zeros((n_out, n_in), np.float32)
    op[np.arange(n_out), lo] += 1.0 - frac
    op[np.arange(n_out), hi] += frac
    return op


def _resize_op(out_hw, in_hw):
    return np.kron(_interp_1d(out_hw[0], in_hw[0]),
                   _interp_1d(out_hw[1], in_hw[1])).astype(np.float32)


def _shift_ops(hw):
    H, W = hw
    n = H * W
    ops = np.zeros((9, n, n), np.float32)
    for kh in (-1, 0, 1):
        for kw in (-1, 0, 1):
            blk = np.zeros((H, W, H, W), np.float32)
            for h in range(max(0, -kh), min(H, H - kh)):
                for w in range(max(0, -kw), min(W, W - kw)):
                    blk[h, w, h + kh, w + kw] = 1.0
            ops[(kh + 1) * 3 + (kw + 1)] = blk.reshape(n, n)
    return ops


def _mss_kernel(*refs):
    (b4_ref, zf_ref, msk_ref, corr_ref, b3_ref, b2_ref, b1_ref,
     rmt_ref, ws_ref, bs_ref, wt_ref, tb_ref) = refs[:12]
    out_ref = refs[-1]
    cref = refs[12:-1]
    convs = [tuple(cref[3 * i:3 * i + 3]) for i in range(9)]
    c_rco, c_b4, c_r3, c_b3, c_r2, c_b2, c_r1, c_b1, c_r0 = convs

    f32 = jnp.float32
    bf16 = jnp.bfloat16
    nb = b4_ref.shape[0]

    def conv(x3, at_r, wv_r, bias_r, relu):
        at = at_r[...]
        nin, nout = at.shape[1], at.shape[2]
        cout = wv_r.shape[0] // 9
        wb = jnp.broadcast_to(wv_r[...], (nb,) + wv_r.shape)
        t3 = jax.lax.dot_general(
            wb, x3, (((2,), (1,)), ((0,), (0,))),
            preferred_element_type=f32).astype(bf16)
        acc = jnp.zeros((nb * cout, nout), f32)
        for k in range(9):
            tk = t3[:, k * cout:(k + 1) * cout, :].reshape(nb * cout, nin)
            acc = acc + jnp.dot(tk, at[k], preferred_element_type=f32)
        acc = acc + bias_r[...]
        if relu:
            acc = jnp.maximum(acc, 0.0)
        return acc

    b4 = b4_ref[...]
    nx, nz = b4.shape[2], zf_ref.shape[2]
    ce = ws_ref.shape[0]
    ws_b = jnp.broadcast_to(ws_ref[...], (nb,) + ws_ref.shape)
    wt_b = jnp.broadcast_to(wt_ref[...], (nb,) + wt_ref.shape)
    xe = (jax.lax.dot_general(ws_b, b4, (((2,), (1,)), ((0,), (0,))),
                              preferred_element_type=f32)
          + bs_ref[...][None])
    ze = (jax.lax.dot_general(wt_b, zf_ref[...], (((2,), (1,)), ((0,), (0,))),
                              preferred_element_type=f32)
          + tb_ref[...][None])
    att = jax.lax.dot_general(
        xe, ze, (((1,), (1,)), ((0,), (0,))),
        preferred_element_type=f32) * (1.0 / math.sqrt(ce))
    att = att - jnp.max(att, axis=-1, keepdims=True)
    p = jnp.exp(att)
    att = p * pl.reciprocal(jnp.sum(p, axis=-1, keepdims=True), approx=True)
    mup = jnp.dot(msk_ref[...], rmt_ref[...],
                  preferred_element_type=f32)
    arn = jnp.sum(att * mup[:, None, :], axis=-1)
    arn = jnp.clip(arn, 0.0, 1.0)
    b4a = (b4.astype(f32) + arn[:, None, :]).astype(bf16)

    def carry(acc_t, cout):
        return acc_t.astype(bf16).reshape(nb, cout, acc_t.shape[1])

    r4 = (conv(corr_ref[...], *c_rco, relu=True)
          + conv(b4a, *c_b4, relu=True))
    r3 = (conv(carry(r4, c_r3[1].shape[1]), *c_r3, relu=True)
          + conv(b3_ref[...], *c_b3, relu=True))
    r2 = (conv(carry(r3, c_r2[1].shape[1]), *c_r2, relu=True)
          + conv(b2_ref[...], *c_b2, relu=True))
    r1 = (conv(carry(r2, c_r1[1].shape[1]), *c_r1, relu=True)
          + conv(b1_ref[...], *c_b1, relu=True))
    mask = conv(carry(r1, c_r0[1].shape[1]), *c_r0, relu=False)
    out_ref[0] = mask.astype(out_ref.dtype)


def kernel(b1, b2, b3, b4, corr, zf_ori, template_mask,
           w_b4, bias_b4, w_b3, bias_b3, w_b2, bias_b2, w_b1, bias_b1,
           w_rCo, bias_rCo, w_r3, bias_r3, w_r2, bias_r2, w_r1, bias_r1,
           w_r0, bias_r0, w_s_embed, bias_s_embed, w_t_embed, bias_t_embed):
    B = b4.shape[0]
    nb = _NB if B % _NB == 0 else (16 if B % 16 == 0 else 1)
    grid = B // nb
    out_hw = (16, 16)
    bf16 = jnp.bfloat16
    f32 = jnp.float32

    hw = lambda t: (int(t.shape[2]), int(t.shape[3]))
    b1_hw, b2_hw, b3_hw, b4_hw = hw(b1), hw(b2), hw(b3), hw(b4)
    corr_hw, zf_hw = hw(corr), hw(zf_ori)
    msk_hw = (int(template_mask.shape[1]), int(template_mask.shape[2]))

    def cm(x):
        Bx, C, H, W = x.shape
        return x.reshape(Bx, C, H * W).astype(bf16)

    b4s, zfs, corrs = cm(b4), cm(zf_ori), cm(corr)
    b3s, b2s, b1s = cm(b3), cm(b2), cm(b1)
    msks = template_mask.reshape(B, msk_hw[0] * msk_hw[1]).astype(f32)

    shift = {s: _shift_ops(s) for s in {b4_hw, b3_hw, b2_hw, b1_hw, out_hw}}
    spatial = {
        'rCo': shift[b4_hw] @ _resize_op(b4_hw, corr_hw),
        'b4':  shift[b4_hw],
        'r3':  shift[b3_hw] @ _resize_op(b3_hw, b4_hw),
        'b3':  shift[b3_hw],
        'r2':  shift[b2_hw] @ _resize_op(b2_hw, b3_hw),
        'b2':  shift[b2_hw],
        'r1':  shift[b1_hw] @ _resize_op(b1_hw, b2_hw),
        'b1':  shift[b1_hw],
        'r0':  shift[out_hw] @ _resize_op(out_hw, b1_hw),
    }
    rm_t = jnp.asarray(_resize_op(zf_hw, msk_hw).T, f32)

    params = {'rCo': (w_rCo, bias_rCo), 'b4': (w_b4, bias_b4),
              'r3': (w_r3, bias_r3), 'b3': (w_b3, bias_b3),
              'r2': (w_r2, bias_r2), 'b2': (w_b2, bias_b2),
              'r1': (w_r1, bias_r1), 'b1': (w_b1, bias_b1),
              'r0': (w_r0, bias_r0)}
    conv_args = []
    for name in ('rCo', 'b4', 'r3', 'b3', 'r2', 'b2', 'r1', 'b1', 'r0'):
        w, bias = params[name]
        cin, cout = int(w.shape[2]), int(w.shape[3])
        wver = (w.reshape(9, cin, cout).transpose(0, 2, 1)
                .reshape(9 * cout, cin).astype(bf16))
        a_t = np.ascontiguousarray(spatial[name].transpose(0, 2, 1))
        conv_args += [jnp.asarray(a_t, bf16),
                      wver,
                      jnp.tile(bias.astype(f32).reshape(cout, 1),
                               (nb, 1))]

    args = [b4s, zfs, msks, corrs, b3s, b2s, b1s,
            rm_t, w_s_embed.astype(bf16).T, bias_s_embed.astype(f32).reshape(-1, 1),
            w_t_embed.astype(bf16).T, bias_t_embed.astype(f32).reshape(-1, 1)] + conv_args

    def batched(shape):
        nz = (0,) * (len(shape) - 1)
        return pl.BlockSpec((nb,) + tuple(shape[1:]),
                            lambda i, _z=nz: (i,) + _z)

    def rep(shape):
        zeros = (0,) * len(shape)
        return pl.BlockSpec(tuple(shape), lambda i, _z=zeros: _z)

    in_specs = ([batched(a.shape) for a in args[:7]]
                + [rep(a.shape) for a in args[7:]])

    n_out = out_hw[0] * out_hw[1]
    cout_final = int(w_r0.shape[3])
    out_shape = jax.ShapeDtypeStruct((grid, nb * cout_final, n_out), f32)

    mask_t = pl.pallas_call(
        _mss_kernel,
        out_shape=out_shape,
        grid=(grid,),
        in_specs=in_specs,
        out_specs=pl.BlockSpec((1, nb * cout_final, n_out),
                               lambda i: (i, 0, 0)),
        compiler_params=pltpu.CompilerParams(
            dimension_semantics=("arbitrary",),
            vmem_limit_bytes=56 * 1024 * 1024),
    )(*args)

    mask = mask_t.reshape(B, cout_final, out_hw[0], out_hw[1])
    return mask
